# baseline (speedup 1.0000x reference)
import functools

import jax
import jax.numpy as jnp
from jax.experimental import pallas as pl
from jax.experimental.pallas import tpu as pltpu

S_BLK = 32


def _rnn_body(idx_ref, emb_ref, wlo_ref, whi_ref, whh_ref, bih_ref,
              bhh_ref, wout_ref, bout_ref, out_ref, tab_ref, xlo_ref,
              xhi_ref, xw_ref, h_ref, acc_ref, psem, *, ns, s_total, b_tot):
    i = pl.program_id(0)
    j = pl.program_id(1)
    bh = h_ref.shape[0]

    @pl.when(j == 0)
    def _():
        cp = pltpu.make_async_copy(emb_ref, tab_ref, psem)
        cp.start()
        cp.wait()
        h_ref[...] = jnp.zeros_like(h_ref)
        acc_ref[...] = jnp.zeros_like(acc_ref)

    base = j * (S_BLK * b_tot) + i * bh
    for t in range(S_BLK):
        for b in range(bh):
            tok = idx_ref[base + t * b_tot + b]
            k = t * bh + b
            w = tab_ref[pl.ds(tok, 1), :]
            xlo_ref[k:k + 1, :] = jax.lax.bitcast_convert_type(
                jnp.left_shift(w, 16), jnp.float32)
            xhi_ref[k:k + 1, :] = jax.lax.bitcast_convert_type(
                w & jnp.int32(-65536), jnp.float32)

    xw_ref[...] = (
        jnp.dot(xlo_ref[...], wlo_ref[...], preferred_element_type=jnp.float32)
        + jnp.dot(xhi_ref[...], whi_ref[...],
                  preferred_element_type=jnp.float32)
        + bih_ref[...]
    )

    h = h_ref[...]
    acc = acc_ref[...]
    whh = whh_ref[...]
    bhh = bhh_ref[...]
    for t in range(S_BLK):
        xw_t = xw_ref[t * bh:(t + 1) * bh, :]
        h = jnp.tanh(xw_t + jnp.dot(h, whh, preferred_element_type=jnp.float32)
                     + bhh)
        acc = acc + h
    h_ref[...] = h
    acc_ref[...] = acc

    @pl.when(j == ns - 1)
    def _():
        z = (jnp.dot(acc, wout_ref[...], preferred_element_type=jnp.float32)
             + s_total * bout_ref[...])
        m = jnp.max(z, axis=1, keepdims=True)
        lse = jnp.log(jnp.sum(jnp.exp(z - m), axis=1, keepdims=True)) + m
        out_ref[...] = z - lse


def kernel(inputs, emb, W_ih, W_hh, b_ih, b_hh, W_out, b_out):
    S, B = inputs.shape
    V, E = emb.shape
    H = W_hh.shape[0]
    C = W_out.shape[0]
    ns = S // S_BLK
    bh = B // 2
    eh = E // 2

    idx = inputs.reshape(-1).astype(jnp.int32)
    tab = jax.lax.bitcast_convert_type(
        emb.astype(jnp.bfloat16).reshape(V, eh, 2), jnp.int32)
    wih_t = W_ih.T

    body = functools.partial(_rnn_body, ns=ns, s_total=float(S), b_tot=B)

    out = pl.pallas_call(
        body,
        out_shape=jax.ShapeDtypeStruct((B, C), jnp.float32),
        grid=(2, ns),
        in_specs=[
            pl.BlockSpec(memory_space=pltpu.SMEM),
            pl.BlockSpec(memory_space=pl.ANY),
            pl.BlockSpec((eh, H), lambda i, j: (0, 0)),
            pl.BlockSpec((eh, H), lambda i, j: (0, 0)),
            pl.BlockSpec((H, H), lambda i, j: (0, 0)),
            pl.BlockSpec((1, H), lambda i, j: (0, 0)),
            pl.BlockSpec((1, H), lambda i, j: (0, 0)),
            pl.BlockSpec((H, C), lambda i, j: (0, 0)),
            pl.BlockSpec((1, C), lambda i, j: (0, 0)),
        ],
        out_specs=pl.BlockSpec((bh, C), lambda i, j: (i, 0)),
        scratch_shapes=[
            pltpu.VMEM((V, eh), jnp.int32),
            pltpu.VMEM((S_BLK * bh, eh), jnp.float32),
            pltpu.VMEM((S_BLK * bh, eh), jnp.float32),
            pltpu.VMEM((S_BLK * bh, H), jnp.float32),
            pltpu.VMEM((bh, H), jnp.float32),
            pltpu.VMEM((bh, H), jnp.float32),
            pltpu.SemaphoreType.DMA,
        ],
        compiler_params=pltpu.CompilerParams(
            dimension_semantics=("parallel", "arbitrary"),
            vmem_limit_bytes=56 * 1024 * 1024,
        ),
        name="rnn_vmem_gather",
    )(
        idx,
        tab,
        wih_t[0::2],
        wih_t[1::2],
        W_hh.T,
        b_ih.reshape(1, H),
        b_hh.reshape(1, H),
        W_out.T,
        b_out.reshape(1, C),
    )
    return out


# speedup vs baseline: 1.4325x; 1.4325x over previous
import functools

import jax
import jax.numpy as jnp
from jax.experimental import pallas as pl
from jax.experimental.pallas import tpu as pltpu

S_BLK = 16
PRE_R = 200


def _bf16_bits(u):
    lsb = jax.lax.shift_right_logical(u, 16) & jnp.int32(1)
    return jax.lax.shift_right_logical(u + jnp.int32(0x7FFF) + lsb, 16)


def _rnn_body(idx_ref, emb_ref, wlo_ref, whi_ref, whh_ref, bih_ref,
              bhh_ref, wout_ref, bout_ref, out_ref, tab_ref, stg_ref,
              xlo_ref, xhi_ref, xw_ref, h_ref, acc_ref, psem,
              *, ns, s_total, b_tot):
    i = pl.program_id(0)
    j = pl.program_id(1)
    bh = h_ref.shape[0]
    v_dim = tab_ref.shape[0]
    nch = v_dim // PRE_R

    @pl.when(j == 0)
    def _():
        def start(c):
            pltpu.make_async_copy(
                emb_ref.at[pl.ds(c * PRE_R, PRE_R), :],
                stg_ref.at[c % 2],
                psem.at[c % 2],
            ).start()

        start(0)
        for c in range(nch):
            if c + 1 < nch:
                start(c + 1)
            pltpu.make_async_copy(
                emb_ref.at[pl.ds(0, PRE_R), :],
                stg_ref.at[c % 2],
                psem.at[c % 2],
            ).wait()
            u = jax.lax.bitcast_convert_type(stg_ref[c % 2], jnp.int32)
            b = _bf16_bits(u)
            rows = pl.ds(c * PRE_R, PRE_R)
            tab_ref[rows, 0:128] = (
                b[:, 0:128]
                | jax.lax.shift_left(b[:, 128:256], 16)
            )
            tab_ref[rows, 128:172] = b[:, 256:300]
        h_ref[...] = jnp.zeros_like(h_ref)
        acc_ref[...] = jnp.zeros_like(acc_ref)

    base = j * (S_BLK * b_tot) + i * bh
    for t in range(S_BLK):
        for b in range(bh):
            tok = idx_ref[base + t * b_tot + b]
            k = t * bh + b
            w = tab_ref[pl.ds(tok, 1), :]
            xlo_ref[k:k + 1, :] = jax.lax.bitcast_convert_type(
                jax.lax.shift_left(w, 16), jnp.float32)
            xhi_ref[k:k + 1, :] = jax.lax.bitcast_convert_type(
                w[:, 0:128] & jnp.int32(-65536), jnp.float32)

    xw_ref[...] = (
        jnp.dot(xlo_ref[...], wlo_ref[...], preferred_element_type=jnp.float32)
        + jnp.dot(xhi_ref[...], whi_ref[...],
                  preferred_element_type=jnp.float32)
        + bih_ref[...]
    )

    h = h_ref[...]
    acc = acc_ref[...]
    whh = whh_ref[...]
    bhh = bhh_ref[...]
    for t in range(S_BLK):
        xw_t = xw_ref[t * bh:(t + 1) * bh, :]
        h = jnp.tanh(xw_t + jnp.dot(h, whh, preferred_element_type=jnp.float32)
                     + bhh)
        acc = acc + h
    h_ref[...] = h
    acc_ref[...] = acc

    @pl.when(j == ns - 1)
    def _():
        z = (jnp.dot(acc, wout_ref[...], preferred_element_type=jnp.float32)
             + s_total * bout_ref[...])
        m = jnp.max(z, axis=1, keepdims=True)
        lse = jnp.log(jnp.sum(jnp.exp(z - m), axis=1, keepdims=True)) + m
        out_ref[...] = z - lse


def kernel(inputs, emb, W_ih, W_hh, b_ih, b_hh, W_out, b_out):
    S, B = inputs.shape
    V, E = emb.shape
    H = W_hh.shape[0]
    C = W_out.shape[0]
    ns = S // S_BLK
    bh = B // 2
    ew = 172

    idx = inputs.reshape(-1).astype(jnp.int32)
    wih_t = W_ih.T
    w_lo = jnp.concatenate([wih_t[0:128], wih_t[256:300]], axis=0)
    w_hi = wih_t[128:256]

    body = functools.partial(_rnn_body, ns=ns, s_total=float(S), b_tot=B)

    out = pl.pallas_call(
        body,
        out_shape=jax.ShapeDtypeStruct((B, C), jnp.float32),
        grid=(2, ns),
        in_specs=[
            pl.BlockSpec(memory_space=pltpu.SMEM),
            pl.BlockSpec(memory_space=pl.ANY),
            pl.BlockSpec((ew, H), lambda i, j: (0, 0)),
            pl.BlockSpec((128, H), lambda i, j: (0, 0)),
            pl.BlockSpec((H, H), lambda i, j: (0, 0)),
            pl.BlockSpec((1, H), lambda i, j: (0, 0)),
            pl.BlockSpec((1, H), lambda i, j: (0, 0)),
            pl.BlockSpec((H, C), lambda i, j: (0, 0)),
            pl.BlockSpec((1, C), lambda i, j: (0, 0)),
        ],
        out_specs=pl.BlockSpec((bh, C), lambda i, j: (i, 0)),
        scratch_shapes=[
            pltpu.VMEM((V, ew), jnp.int32),
            pltpu.VMEM((2, PRE_R, E), jnp.float32),
            pltpu.VMEM((S_BLK * bh, ew), jnp.float32),
            pltpu.VMEM((S_BLK * bh, 128), jnp.float32),
            pltpu.VMEM((S_BLK * bh, H), jnp.float32),
            pltpu.VMEM((bh, H), jnp.float32),
            pltpu.VMEM((bh, H), jnp.float32),
            pltpu.SemaphoreType.DMA((2,)),
        ],
        compiler_params=pltpu.CompilerParams(
            dimension_semantics=("parallel", "arbitrary"),
            vmem_limit_bytes=58 * 1024 * 1024,
        ),
        name="rnn_vmem_gather",
    )(
        idx,
        emb,
        w_lo,
        w_hi,
        W_hh.T,
        b_ih.reshape(1, H),
        b_hh.reshape(1, H),
        W_out.T,
        b_out.reshape(1, C),
    )
    return out


# speedup vs baseline: 2.4808x; 1.7318x over previous
import functools

import jax
import jax.numpy as jnp
from jax.experimental import pallas as pl
from jax.experimental.pallas import tpu as pltpu

S_BLK = 32


def _rnn_body(idx_ref, emb_ref, wih_ref, whh_ref, bih_ref, bhh_ref,
              wout_ref, bout_ref, out_ref, xw_ref, h_ref, acc_ref,
              xbuf_ref, gsem, *, ns, s_total, b_tot):
    i = pl.program_id(0)
    j = pl.program_id(1)
    bh = h_ref.shape[0]
    n_rows = S_BLK * bh

    jb = jnp.minimum(j, ns - 1)
    base = jb * (S_BLK * b_tot) + i * bh
    slot_g = jax.lax.rem(j, 2)
    slot_c = jax.lax.rem(j + 1, 2)

    @pl.when(j >= 1)
    def _():
        for k in range(n_rows):
            pltpu.make_async_copy(
                emb_ref.at[pl.ds(0, 1), :],
                xbuf_ref.at[slot_c, k],
                gsem.at[slot_c],
            ).wait()

    e_dim = emb_ref.shape[1]
    xw_ref[...] = (
        jnp.dot(xbuf_ref[slot_c].reshape(n_rows, e_dim), wih_ref[...],
                preferred_element_type=jnp.float32)
        + bih_ref[...]
    )

    @pl.when(j < ns)
    def _():
        for t in range(S_BLK):
            for b in range(bh):
                tok = idx_ref[base + t * b_tot + b]
                pltpu.make_async_copy(
                    emb_ref.at[pl.ds(tok, 1), :],
                    xbuf_ref.at[slot_g, t * bh + b],
                    gsem.at[slot_g],
                ).start()

    h = h_ref[...]
    acc = acc_ref[...]
    whh = whh_ref[...]
    bhh = bhh_ref[...]
    for t in range(S_BLK):
        xw_t = xw_ref[t * bh:(t + 1) * bh, :]
        h = jnp.tanh(xw_t + jnp.dot(h, whh, preferred_element_type=jnp.float32)
                     + bhh)
        acc = acc + h
    h_ref[...] = h
    acc_ref[...] = acc

    @pl.when(j == 0)
    def _():
        h_ref[...] = jnp.zeros_like(h_ref)
        acc_ref[...] = jnp.zeros_like(acc_ref)

    @pl.when(j == ns)
    def _():
        z = (jnp.dot(acc, wout_ref[...], preferred_element_type=jnp.float32)
             + s_total * bout_ref[...])
        m = jnp.max(z, axis=1, keepdims=True)
        lse = jnp.log(jnp.sum(jnp.exp(z - m), axis=1, keepdims=True)) + m
        out_ref[...] = z - lse


def kernel(inputs, emb, W_ih, W_hh, b_ih, b_hh, W_out, b_out):
    S, B = inputs.shape
    V, E = emb.shape
    H = W_hh.shape[0]
    C = W_out.shape[0]
    ns = S // S_BLK
    bh = B // 2

    idx = inputs.reshape(-1).astype(jnp.int32)

    body = functools.partial(_rnn_body, ns=ns, s_total=float(S), b_tot=B)

    out = pl.pallas_call(
        body,
        out_shape=jax.ShapeDtypeStruct((B, C), jnp.float32),
        grid=(2, ns + 1),
        in_specs=[
            pl.BlockSpec(memory_space=pltpu.SMEM),
            pl.BlockSpec(memory_space=pl.ANY),
            pl.BlockSpec((E, H), lambda i, j: (0, 0)),
            pl.BlockSpec((H, H), lambda i, j: (0, 0)),
            pl.BlockSpec((1, H), lambda i, j: (0, 0)),
            pl.BlockSpec((1, H), lambda i, j: (0, 0)),
            pl.BlockSpec((H, C), lambda i, j: (0, 0)),
            pl.BlockSpec((1, C), lambda i, j: (0, 0)),
        ],
        out_specs=pl.BlockSpec((bh, C), lambda i, j: (i, 0)),
        scratch_shapes=[
            pltpu.VMEM((S_BLK * bh, H), jnp.float32),
            pltpu.VMEM((bh, H), jnp.float32),
            pltpu.VMEM((bh, H), jnp.float32),
            pltpu.VMEM((2, S_BLK * bh, 1, E), jnp.float32),
            pltpu.SemaphoreType.DMA((2,)),
        ],
        compiler_params=pltpu.CompilerParams(
            dimension_semantics=("parallel", "arbitrary"),
        ),
        name="rnn_fused_gather",
    )(
        idx,
        emb,
        W_ih.T,
        W_hh.T,
        b_ih.reshape(1, H),
        b_hh.reshape(1, H),
        W_out.T,
        b_out.reshape(1, C),
    )
    return out
